# Initial kernel scaffold; baseline (speedup 1.0000x reference)
import jax
import jax.numpy as jnp
from jax.experimental import pallas as pl


def kernel(x, edge_index, W1, b1, W2, b2):
    raise NotImplementedError("write your pallas kernel here")


# speedup vs baseline: 5.4512x; 5.4512x over previous
import functools

import jax
import jax.numpy as jnp
from jax import lax
from jax.experimental import pallas as pl
from jax.experimental.pallas import tpu as pltpu
from jax.experimental.pallas import tpu_sc as plsc

N = 10000
E = 160000
NP = 10240
EP = 163840
CH = 128
NCH = EP // (32 * CH)
ROWS_PER_TILE = NP // 16

_mesh = plsc.VectorSubcoreMesh(core_axis_name="c", subcore_axis_name="s")


def _zero_fill(buf, nrows, ncols):
    def row(i, _):
        def col(j, _):
            buf[i, pl.ds(j * 16, 16)] = jnp.zeros((16,), jnp.float32)
            return 0
        return lax.fori_loop(0, ncols // 16, col, 0)
    lax.fori_loop(0, nrows, row, 0)


def _deg_call(dst):
    @functools.partial(
        pl.kernel,
        out_type=jax.ShapeDtypeStruct((2, NP, 16), jnp.float32),
        mesh=_mesh,
        scratch_types=[
            pltpu.VMEM((CH,), jnp.int32),
            pltpu.VMEM((CH, 16), jnp.float32),
            pltpu.VMEM_SHARED((NP, 16), jnp.float32),
        ],
    )
    def deg_kernel(dst_hbm, out_hbm, didx, buf, acc):
        c = lax.axis_index("c")
        s = lax.axis_index("s")
        _zero_fill(buf, CH, 16)
        for b in range(ROWS_PER_TILE // CH):
            pltpu.sync_copy(buf, acc.at[pl.ds(s * ROWS_PER_TILE + b * CH, CH), :])
        plsc.subcore_barrier()
        def fill_one(i, _):
            buf[i] = jnp.ones((16,), jnp.float32)
            return 0
        lax.fori_loop(0, CH, fill_one, 0)
        base = (c * 16 + s) * (EP // 32)
        def chunk(i, _):
            pltpu.sync_copy(dst_hbm.at[pl.ds(base + i * CH, CH)], didx)
            pltpu.sync_copy(buf, acc.at[didx], add=True)
            return 0
        lax.fori_loop(0, NCH, chunk, 0)
        plsc.subcore_barrier()
        for b in range(ROWS_PER_TILE // CH):
            r0 = s * ROWS_PER_TILE + b * CH
            pltpu.sync_copy(acc.at[pl.ds(r0, CH), :], buf)
            pltpu.sync_copy(buf, out_hbm.at[c, pl.ds(r0, CH), :])

    return deg_kernel(dst)


def _prop_call(g, src, dst):
    F = 128
    @functools.partial(
        pl.kernel,
        out_type=jax.ShapeDtypeStruct((2, NP, F), jnp.float32),
        mesh=_mesh,
        scratch_types=[
            pltpu.VMEM((CH,), jnp.int32),
            pltpu.VMEM((CH,), jnp.int32),
            pltpu.VMEM((CH, F), jnp.float32),
            pltpu.VMEM_SHARED((NP, F), jnp.float32),
            pltpu.SemaphoreType.DMA,
        ],
    )
    def prop_kernel(g_hbm, src_hbm, dst_hbm, out_hbm, sidx, didx, rows, acc, sem):
        c = lax.axis_index("c")
        s = lax.axis_index("s")
        _zero_fill(rows, CH, F)
        for b in range(ROWS_PER_TILE // CH):
            pltpu.sync_copy(rows, acc.at[pl.ds(s * ROWS_PER_TILE + b * CH, CH), :])
        plsc.subcore_barrier()
        base = (c * 16 + s) * (EP // 32)
        def chunk(i, _):
            e0 = base + i * CH
            pltpu.sync_copy(src_hbm.at[pl.ds(e0, CH)], sidx)
            pltpu.async_copy(g_hbm.at[sidx], rows, sem).wait()
            pltpu.sync_copy(dst_hbm.at[pl.ds(e0, CH)], didx)
            pltpu.sync_copy(rows, acc.at[didx], add=True)
            return 0
        lax.fori_loop(0, NCH, chunk, 0)
        plsc.subcore_barrier()
        for b in range(ROWS_PER_TILE // CH):
            r0 = s * ROWS_PER_TILE + b * CH
            pltpu.sync_copy(acc.at[pl.ds(r0, CH), :], rows)
            pltpu.sync_copy(rows, out_hbm.at[c, pl.ds(r0, CH), :])

    return prop_kernel(g, src, dst)


_BM = 1024


def _dis_block(degb):
    return lax.rsqrt(degb[0, :, :1] + degb[1, :, :1] + 1.0)


def _mm1_call(xp, w1p, deg16):
    def body(xb, wb, degb, ob):
        dis = _dis_block(degb)
        acc = jnp.dot(xb[...], wb[...], preferred_element_type=jnp.float32,
                      precision=lax.Precision.HIGHEST)
        ob[...] = acc * dis

    return pl.pallas_call(
        body,
        grid=(NP // _BM,),
        in_specs=[
            pl.BlockSpec((_BM, 972), lambda i: (i, 0)),
            pl.BlockSpec((972, 128), lambda i: (0, 0)),
            pl.BlockSpec((2, _BM, 16), lambda i: (0, i, 0)),
        ],
        out_specs=pl.BlockSpec((_BM, 128), lambda i: (i, 0)),
        out_shape=jax.ShapeDtypeStruct((NP, 128), jnp.float32),
    )(xp, w1p, deg16)


def _mm2_call(deg16, s1, g1, b1r, w2p):
    def body(degb, sb, gb, bb, wb, ob):
        dis = _dis_block(degb)
        h = jnp.maximum((sb[0] + sb[1] + gb[...]) * dis + bb[...], 0.0)
        acc = jnp.dot(h, wb[...], preferred_element_type=jnp.float32,
                      precision=lax.Precision.HIGHEST)
        ob[...] = acc * dis

    return pl.pallas_call(
        body,
        grid=(NP // _BM,),
        in_specs=[
            pl.BlockSpec((2, _BM, 16), lambda i: (0, i, 0)),
            pl.BlockSpec((2, _BM, 128), lambda i: (0, i, 0)),
            pl.BlockSpec((_BM, 128), lambda i: (i, 0)),
            pl.BlockSpec((1, 128), lambda i: (0, 0)),
            pl.BlockSpec((128, 128), lambda i: (0, 0)),
        ],
        out_specs=pl.BlockSpec((_BM, 128), lambda i: (i, 0)),
        out_shape=jax.ShapeDtypeStruct((NP, 128), jnp.float32),
    )(deg16, s1, g1, b1r, w2p)


def _final_call(deg16, s2, g2, b2r):
    def body(degb, sb, gb, bb, ob):
        dis = _dis_block(degb)
        ob[...] = (sb[0, :, :64] + sb[1, :, :64] + gb[:, :64]) * dis + bb[...]

    return pl.pallas_call(
        body,
        grid=(NP // _BM,),
        in_specs=[
            pl.BlockSpec((2, _BM, 16), lambda i: (0, i, 0)),
            pl.BlockSpec((2, _BM, 128), lambda i: (0, i, 0)),
            pl.BlockSpec((_BM, 128), lambda i: (i, 0)),
            pl.BlockSpec((1, 64), lambda i: (0, 0)),
        ],
        out_specs=pl.BlockSpec((_BM, 64), lambda i: (i, 0)),
        out_shape=jax.ShapeDtypeStruct((NP, 64), jnp.float32),
    )(deg16, s2, g2, b2r)


def kernel(x, edge_index, W1, b1, W2, b2):
    ei = edge_index.astype(jnp.int32)
    pad_idx = jnp.full((EP - E,), N, jnp.int32)
    src = jnp.concatenate([ei[0], pad_idx])
    dst = jnp.concatenate([ei[1], pad_idx])
    xp = jnp.pad(x, ((0, NP - N), (0, 0)))
    w1p = jnp.pad(W1, ((972 - 384, 0), (0, 0)))

    w2p = jnp.pad(W2, ((0, 0), (0, 64)))

    deg16 = _deg_call(dst)
    g1 = _mm1_call(xp, w1p, deg16)
    s1 = _prop_call(g1, src, dst)
    g2 = _mm2_call(deg16, s1, g1, b1.reshape(1, 128), w2p)
    s2 = _prop_call(g2, src, dst)
    z = _final_call(deg16, s2, g2, b2.reshape(1, 64))
    return z[:N]


# speedup vs baseline: 7.3204x; 1.3429x over previous
import functools

import jax
import jax.numpy as jnp
from jax import lax
from jax.experimental import pallas as pl
from jax.experimental.pallas import tpu as pltpu
from jax.experimental.pallas import tpu_sc as plsc

N = 10000
E = 160000
NP = 10240
EP = 163840
CH = 128
NCH = EP // (32 * CH)
ROWS_PER_TILE = NP // 16

_mesh = plsc.VectorSubcoreMesh(core_axis_name="c", subcore_axis_name="s")


def _zero_fill(buf, nrows, ncols):
    def row(i, _):
        def col(j, _):
            buf[i, pl.ds(j * 16, 16)] = jnp.zeros((16,), jnp.float32)
            return 0
        return lax.fori_loop(0, ncols // 16, col, 0)
    lax.fori_loop(0, nrows, row, 0)


def _deg_call(dst2d):
    @functools.partial(
        pl.kernel,
        out_type=jax.ShapeDtypeStruct((2, NP, 16), jnp.float32),
        mesh=_mesh,
        scratch_types=[
            pltpu.VMEM((NCH, CH), jnp.int32),
            pltpu.VMEM((CH, 16), jnp.float32),
            pltpu.VMEM_SHARED((NP, 16), jnp.float32),
        ],
    )
    def deg_kernel(dst_hbm, out_hbm, didx, buf, acc):
        c = lax.axis_index("c")
        s = lax.axis_index("s")
        wid = c * 16 + s
        _zero_fill(buf, CH, 16)
        for b in range(ROWS_PER_TILE // CH):
            pltpu.sync_copy(buf, acc.at[pl.ds(s * ROWS_PER_TILE + b * CH, CH), :])
        plsc.subcore_barrier()
        pltpu.sync_copy(dst_hbm.at[pl.ds(wid * NCH, NCH), :], didx)
        def fill_one(i, _):
            buf[i] = jnp.ones((16,), jnp.float32)
            return 0
        lax.fori_loop(0, CH, fill_one, 0)
        def chunk(i, _):
            pltpu.sync_copy(buf, acc.at[didx.at[i]], add=True)
            return 0
        lax.fori_loop(0, NCH, chunk, 0)
        plsc.subcore_barrier()
        for b in range(ROWS_PER_TILE // CH):
            r0 = s * ROWS_PER_TILE + b * CH
            pltpu.sync_copy(acc.at[pl.ds(r0, CH), :], buf)
            pltpu.sync_copy(buf, out_hbm.at[c, pl.ds(r0, CH), :])

    return deg_kernel(dst2d)


def _prop_call(g, src2d, dst2d):
    F = 128

    @functools.partial(
        pl.kernel,
        out_type=jax.ShapeDtypeStruct((2, NP, F), jnp.float32),
        mesh=_mesh,
        scratch_types=[
            pltpu.VMEM((NCH, CH), jnp.int32),
            pltpu.VMEM((NCH, CH), jnp.int32),
            pltpu.VMEM((CH, F), jnp.float32),
            pltpu.VMEM((CH, F), jnp.float32),
            pltpu.VMEM_SHARED((NP, F), jnp.float32),
            pltpu.SemaphoreType.DMA,
            pltpu.SemaphoreType.DMA,
        ],
    )
    def prop_kernel(g_hbm, src_hbm, dst_hbm, out_hbm,
                    sidx, didx, rows0, rows1, acc, gsem0, gsem1):
        c = lax.axis_index("c")
        s = lax.axis_index("s")
        wid = c * 16 + s
        _zero_fill(rows0, CH, F)
        for b in range(ROWS_PER_TILE // CH):
            pltpu.sync_copy(rows0, acc.at[pl.ds(s * ROWS_PER_TILE + b * CH, CH), :])
        plsc.subcore_barrier()
        pltpu.sync_copy(src_hbm.at[pl.ds(wid * NCH, NCH), :], sidx)
        pltpu.sync_copy(dst_hbm.at[pl.ds(wid * NCH, NCH), :], didx)
        pltpu.async_copy(g_hbm.at[sidx.at[0]], rows0, gsem0)
        def step(j, _):
            i0 = 2 * j
            i1 = i0 + 1
            pltpu.make_async_copy(g_hbm.at[sidx.at[i0]], rows0, gsem0).wait()
            pltpu.async_copy(g_hbm.at[sidx.at[i1]], rows1, gsem1)
            pltpu.sync_copy(rows0, acc.at[didx.at[i0]], add=True)
            pltpu.make_async_copy(g_hbm.at[sidx.at[i1]], rows1, gsem1).wait()
            @pl.when(j < NCH // 2 - 1)
            def _():
                pltpu.async_copy(g_hbm.at[sidx.at[i0 + 2]], rows0, gsem0)
            pltpu.sync_copy(rows1, acc.at[didx.at[i1]], add=True)
            return 0
        lax.fori_loop(0, NCH // 2, step, 0)
        plsc.subcore_barrier()
        for b in range(ROWS_PER_TILE // CH):
            r0 = s * ROWS_PER_TILE + b * CH
            pltpu.sync_copy(acc.at[pl.ds(r0, CH), :], rows0)
            pltpu.sync_copy(rows0, out_hbm.at[c, pl.ds(r0, CH), :])

    return prop_kernel(g, src2d, dst2d)


_BM = 1000


def _dis_block(degb):
    return lax.rsqrt(degb[0, :, :1] + degb[1, :, :1] + 1.0)


def _mm1_call(x, w1p, deg16):
    def body(xb, wb, degb, ob):
        dis = _dis_block(degb)
        acc = jnp.dot(xb[...], wb[...], preferred_element_type=jnp.float32,
                      precision=lax.Precision.HIGHEST)
        ob[...] = acc * dis

    return pl.pallas_call(
        body,
        grid=(N // _BM,),
        in_specs=[
            pl.BlockSpec((_BM, 972), lambda i: (i, 0)),
            pl.BlockSpec((972, 128), lambda i: (0, 0)),
            pl.BlockSpec((2, _BM, 16), lambda i: (0, i, 0)),
        ],
        out_specs=pl.BlockSpec((_BM, 128), lambda i: (i, 0)),
        out_shape=jax.ShapeDtypeStruct((NP, 128), jnp.float32),
    )(x, w1p, deg16)


def _mm2_call(deg16, s1, g1, b1r, w2p):
    def body(degb, sb, gb, bb, wb, ob):
        dis = _dis_block(degb)
        h = jnp.maximum((sb[0] + sb[1] + gb[...]) * dis + bb[...], 0.0)
        acc = jnp.dot(h, wb[...], preferred_element_type=jnp.float32,
                      precision=lax.Precision.HIGHEST)
        ob[...] = acc * dis

    return pl.pallas_call(
        body,
        grid=(N // _BM,),
        in_specs=[
            pl.BlockSpec((2, _BM, 16), lambda i: (0, i, 0)),
            pl.BlockSpec((2, _BM, 128), lambda i: (0, i, 0)),
            pl.BlockSpec((_BM, 128), lambda i: (i, 0)),
            pl.BlockSpec((1, 128), lambda i: (0, 0)),
            pl.BlockSpec((128, 128), lambda i: (0, 0)),
        ],
        out_specs=pl.BlockSpec((_BM, 128), lambda i: (i, 0)),
        out_shape=jax.ShapeDtypeStruct((NP, 128), jnp.float32),
    )(deg16, s1, g1, b1r, w2p)


def _final_call(deg16, s2, g2, b2r):
    def body(degb, sb, gb, bb, ob):
        dis = _dis_block(degb)
        ob[...] = (sb[0, :, :64] + sb[1, :, :64] + gb[:, :64]) * dis + bb[...]

    return pl.pallas_call(
        body,
        grid=(N // _BM,),
        in_specs=[
            pl.BlockSpec((2, _BM, 16), lambda i: (0, i, 0)),
            pl.BlockSpec((2, _BM, 128), lambda i: (0, i, 0)),
            pl.BlockSpec((_BM, 128), lambda i: (i, 0)),
            pl.BlockSpec((1, 64), lambda i: (0, 0)),
        ],
        out_specs=pl.BlockSpec((_BM, 64), lambda i: (i, 0)),
        out_shape=jax.ShapeDtypeStruct((N, 64), jnp.float32),
    )(deg16, s2, g2, b2r)


def kernel(x, edge_index, W1, b1, W2, b2):
    ei = edge_index.astype(jnp.int32)
    pad_idx = jnp.full((EP - E,), N, jnp.int32)
    src2d = jnp.concatenate([ei[0], pad_idx]).reshape(EP // CH, CH)
    dst2d = jnp.concatenate([ei[1], pad_idx]).reshape(EP // CH, CH)
    w1p = jnp.pad(W1, ((972 - 384, 0), (0, 0)))
    w2p = jnp.pad(W2, ((0, 0), (0, 64)))

    deg16 = _deg_call(dst2d)
    g1 = _mm1_call(x, w1p, deg16)
    s1 = _prop_call(g1, src2d, dst2d)
    g2 = _mm2_call(deg16, s1, g1, b1.reshape(1, 128), w2p)
    s2 = _prop_call(g2, src2d, dst2d)
    return _final_call(deg16, s2, g2, b2.reshape(1, 64))
